# baseline (speedup 1.0000x reference)
import functools

import jax
import jax.numpy as jnp
from jax import lax
from jax.experimental import pallas as pl
from jax.experimental.pallas import tpu as pltpu
from jax.experimental.pallas import tpu_sc as plsc

B, H, W = 16, 512, 512
N = B * H * W
NC, NS, L = 2, 16, 16
NW = NC * NS
PER_W = N // NW
CHUNK = 16384
NCHUNK = PER_W // CHUNK
GROUPS = CHUNK // 64

_mesh = plsc.VectorSubcoreMesh(core_axis_name="c", subcore_axis_name="s")


@functools.partial(
    pl.kernel,
    out_type=jax.ShapeDtypeStruct((NW, 2, L), jnp.float32),
    mesh=_mesh,
    scratch_types=[
        pltpu.VMEM((CHUNK,), jnp.float32),
        pltpu.VMEM((CHUNK,), jnp.float32),
        pltpu.VMEM((CHUNK,), jnp.float32),
        pltpu.VMEM((CHUNK,), jnp.float32),
        pltpu.VMEM((CHUNK // 4,), jnp.int32),
        pltpu.VMEM((CHUNK // 4,), jnp.int32),
        pltpu.VMEM((2, L), jnp.float32),
        pltpu.SemaphoreType.DMA,
        pltpu.SemaphoreType.DMA,
    ],
)
def _sc_loss(x_hbm, t_hbm, m_hbm, out_hbm,
             xb0, xb1, tb0, tb1, mb0, mb1, res, sem0, sem1):
    wid = lax.axis_index("s") * NC + lax.axis_index("c")
    base = wid * PER_W
    xbs, tbs, mbs, sems = (xb0, xb1), (tb0, tb1), (mb0, mb1), (sem0, sem1)

    def start(g):
        slot = g % 2
        off = pl.multiple_of(base + g * CHUNK, CHUNK)
        offw = pl.multiple_of((base + g * CHUNK) // 4, CHUNK // 4)
        return [
            pltpu.async_copy(x_hbm.at[pl.ds(off, CHUNK)], xbs[slot], sems[slot]),
            pltpu.async_copy(t_hbm.at[pl.ds(off, CHUNK)], tbs[slot], sems[slot]),
            pltpu.async_copy(m_hbm.at[pl.ds(offw, CHUNK // 4)], mbs[slot], sems[slot]),
        ]

    acc = jnp.zeros((L,), jnp.float32)
    cnt = jnp.zeros((L,), jnp.float32)

    pend = start(0)
    for g in range(NCHUNK):
        nxt = start(g + 1) if g + 1 < NCHUNK else None
        for c in pend:
            c.wait()
        xb, tb, mb = xbs[g % 2], tbs[g % 2], mbs[g % 2]

        def body(i, carry, xb=xb, tb=tb, mb=mb):
            a, n = carry
            w = mb[pl.ds(i * 16, 16)]
            goff = i * 64
            for b in range(4):
                xg = xb[pl.ds(goff + 16 * b, 16)]
                tg = tb[pl.ds(goff + 16 * b, 16)]
                d = xg - tg
                ad = jnp.abs(d)
                c1 = jnp.minimum(ad, 1.0)
                loss = c1 * (ad - 0.5 * c1)
                mf = ((w >> (8 * b)) & 1).astype(jnp.float32)
                a = a + loss * mf
                n = n + mf
            return a, n

        acc, cnt = lax.fori_loop(0, GROUPS, body, (acc, cnt))
        pend = nxt

    res[0, :] = acc
    res[1, :] = cnt
    pltpu.sync_copy(res, out_hbm.at[wid])


def kernel(inputs, targets, mask):
    x = inputs.reshape(N)
    t = targets.reshape(N)
    mt = mask.reshape(N // 64, 4, L).swapaxes(1, 2)
    m32 = mt.view(jnp.int32).reshape(N // 4)
    out = _sc_loss(x, t, m32)
    s = jnp.sum(out[:, 0, :])
    c = jnp.sum(out[:, 1, :])
    return s / jnp.maximum(c, 1.0)


# speedup vs baseline: 4.1356x; 4.1356x over previous
import functools

import jax
import jax.numpy as jnp
from jax import lax
from jax.experimental import pallas as pl
from jax.experimental.pallas import tpu as pltpu
from jax.experimental.pallas import tpu_sc as plsc

B, H, W = 16, 512, 512
N = B * H * W
NC, NS, L = 2, 16, 16
NW = NC * NS
PER_W = N // NW
CHUNK = 16384
NCHUNK = PER_W // CHUNK
GROUPS = CHUNK // 512

_mesh = plsc.VectorSubcoreMesh(core_axis_name="c", subcore_axis_name="s")


@functools.partial(
    pl.kernel,
    out_type=jax.ShapeDtypeStruct((NW, 2, L), jnp.float32),
    mesh=_mesh,
    scratch_types=[
        pltpu.VMEM((CHUNK,), jnp.float32),
        pltpu.VMEM((CHUNK,), jnp.float32),
        pltpu.VMEM((CHUNK,), jnp.float32),
        pltpu.VMEM((CHUNK,), jnp.float32),
        pltpu.VMEM((CHUNK // 32,), jnp.uint32),
        pltpu.VMEM((CHUNK // 32,), jnp.uint32),
        pltpu.VMEM((2, L), jnp.float32),
        pltpu.SemaphoreType.DMA,
        pltpu.SemaphoreType.DMA,
    ],
)
def _sc_loss(x_hbm, t_hbm, m_hbm, out_hbm,
             xb0, xb1, tb0, tb1, mb0, mb1, res, sem0, sem1):
    wid = lax.axis_index("s") * NC + lax.axis_index("c")
    base = wid * PER_W
    xbs, tbs, mbs, sems = (xb0, xb1), (tb0, tb1), (mb0, mb1), (sem0, sem1)

    def start(g):
        slot = g % 2
        off = pl.multiple_of(base + g * CHUNK, CHUNK)
        offw = pl.multiple_of((base + g * CHUNK) // 32, CHUNK // 32)
        return [
            pltpu.async_copy(x_hbm.at[pl.ds(off, CHUNK)], xbs[slot], sems[slot]),
            pltpu.async_copy(t_hbm.at[pl.ds(off, CHUNK)], tbs[slot], sems[slot]),
            pltpu.async_copy(m_hbm.at[pl.ds(offw, CHUNK // 32)], mbs[slot], sems[slot]),
        ]

    acc = jnp.zeros((L,), jnp.float32)
    cnt = jnp.zeros((L,), jnp.float32)

    pend = start(0)
    for g in range(NCHUNK):
        nxt = start(g + 1) if g + 1 < NCHUNK else None
        for c in pend:
            c.wait()
        xb, tb, mb = xbs[g % 2], tbs[g % 2], mbs[g % 2]

        def body(i, carry, xb=xb, tb=tb, mb=mb):
            a, n = carry
            w = mb[pl.ds(i * 16, 16)]
            goff = i * 512
            for b in range(32):
                xg = xb[pl.ds(goff + 16 * b, 16)]
                tg = tb[pl.ds(goff + 16 * b, 16)]
                d = xg - tg
                ad = jnp.abs(d)
                c1 = jnp.minimum(ad, 1.0)
                loss = c1 * (ad - 0.5 * c1)
                mf = ((w >> jnp.uint32(b)) & jnp.uint32(1)).astype(jnp.float32)
                a = a + loss * mf
                n = n + mf
            return a, n

        acc, cnt = lax.fori_loop(0, GROUPS, body, (acc, cnt))
        pend = nxt

    res[0, :] = acc
    res[1, :] = cnt
    pltpu.sync_copy(res, out_hbm.at[wid])


def kernel(inputs, targets, mask):
    x = inputs.reshape(N)
    t = targets.reshape(N)
    mm = mask.reshape(N // 512, 32, L).astype(jnp.uint32)
    mw = jnp.sum(mm << jnp.arange(32, dtype=jnp.uint32)[None, :, None], axis=1)
    out = _sc_loss(x, t, mw.reshape(N // 32))
    s = jnp.sum(out[:, 0, :])
    c = jnp.sum(out[:, 1, :])
    return s / jnp.maximum(c, 1.0)


# speedup vs baseline: 5.4923x; 1.3281x over previous
import functools

import jax
import jax.numpy as jnp
from jax import lax
from jax.experimental import pallas as pl
from jax.experimental.pallas import tpu as pltpu
from jax.experimental.pallas import tpu_sc as plsc

B, H, W = 16, 512, 512
N = B * H * W
NC, NS, L = 2, 16, 16
NW = NC * NS
PER_W = N // NW
CHUNK = 16384
NCHUNK = PER_W // CHUNK
GROUPS = CHUNK // 512

_mesh = plsc.VectorSubcoreMesh(core_axis_name="c", subcore_axis_name="s")


@functools.partial(
    pl.kernel,
    out_type=jax.ShapeDtypeStruct((NW, 2, L), jnp.float32),
    mesh=_mesh,
    scratch_types=[
        pltpu.VMEM((CHUNK,), jnp.float32),
        pltpu.VMEM((CHUNK,), jnp.float32),
        pltpu.VMEM((CHUNK,), jnp.float32),
        pltpu.VMEM((CHUNK,), jnp.float32),
        pltpu.VMEM((CHUNK // 32,), jnp.uint32),
        pltpu.VMEM((CHUNK // 32,), jnp.uint32),
        pltpu.VMEM((2, L), jnp.float32),
        pltpu.SemaphoreType.DMA,
        pltpu.SemaphoreType.DMA,
    ],
)
def _sc_loss(x_hbm, t_hbm, m_hbm, out_hbm,
             xb0, xb1, tb0, tb1, mb0, mb1, res, sem0, sem1):
    wid = lax.axis_index("s") * NC + lax.axis_index("c")
    base = wid * PER_W
    xbs, tbs, mbs, sems = (xb0, xb1), (tb0, tb1), (mb0, mb1), (sem0, sem1)

    def start(g):
        slot = g % 2
        off = pl.multiple_of(base + g * CHUNK, CHUNK)
        offw = pl.multiple_of((base + g * CHUNK) // 32, CHUNK // 32)
        return [
            pltpu.async_copy(x_hbm.at[pl.ds(off, CHUNK)], xbs[slot], sems[slot]),
            pltpu.async_copy(t_hbm.at[pl.ds(off, CHUNK)], tbs[slot], sems[slot]),
            pltpu.async_copy(m_hbm.at[pl.ds(offw, CHUNK // 32)], mbs[slot], sems[slot]),
        ]

    acc = jnp.zeros((L,), jnp.float32)
    cnt = jnp.zeros((L,), jnp.float32)

    pend = start(0)
    for g in range(NCHUNK):
        nxt = start(g + 1) if g + 1 < NCHUNK else None
        for c in pend:
            c.wait()
        xb, tb, mb = xbs[g % 2], tbs[g % 2], mbs[g % 2]

        def body(i, carry, xb=xb, tb=tb, mb=mb):
            a, n = carry
            i2 = i >> 3
            r16 = (i & 7) * 16
            w = mb[pl.ds(i2 * 128 + r16, 16)]
            goff = i2 * 4096 + r16
            for b in range(32):
                xg = xb[pl.ds(goff + 128 * b, 16)]
                tg = tb[pl.ds(goff + 128 * b, 16)]
                d = xg - tg
                ad = jnp.abs(d)
                c1 = jnp.minimum(ad, 1.0)
                loss = c1 * (ad - 0.5 * c1)
                mf = ((w >> jnp.uint32(b)) & jnp.uint32(1)).astype(jnp.float32)
                a = a + loss * mf
                n = n + mf
            return a, n

        acc, cnt = lax.fori_loop(0, GROUPS, body, (acc, cnt))
        pend = nxt

    res[0, :] = acc
    res[1, :] = cnt
    pltpu.sync_copy(res, out_hbm.at[wid])


def kernel(inputs, targets, mask):
    x = inputs.reshape(N)
    t = targets.reshape(N)
    mm = mask.reshape(N // 4096, 32, 128).astype(jnp.uint32)
    mw = jnp.sum(mm << jnp.arange(32, dtype=jnp.uint32)[None, :, None], axis=1)
    out = _sc_loss(x, t, mw.reshape(N // 32))
    s = jnp.sum(out[:, 0, :])
    c = jnp.sum(out[:, 1, :])
    return s / jnp.maximum(c, 1.0)


# speedup vs baseline: 6.9605x; 1.2673x over previous
import functools

import jax
import jax.numpy as jnp
from jax import lax
from jax.experimental import pallas as pl
from jax.experimental.pallas import tpu as pltpu
from jax.experimental.pallas import tpu_sc as plsc

B, H, W = 16, 512, 512
N = B * H * W
ROWS = N // W
NC, NS, L = 2, 16, 16
NW = NC * NS

SC_CHUNKS = 2
SC_ROWS = NW * SC_CHUNKS * 32
TC_ROWS = ROWS - SC_ROWS
ROWS_PER_W = SC_ROWS // NW
GROUPS = 32

_mesh = plsc.VectorSubcoreMesh(core_axis_name="c", subcore_axis_name="s")


@functools.partial(
    pl.kernel,
    out_type=jax.ShapeDtypeStruct((NW, 2, L), jnp.float32),
    mesh=_mesh,
    scratch_types=[
        pltpu.VMEM((32, W), jnp.float32),
        pltpu.VMEM((32, W), jnp.float32),
        pltpu.VMEM((32, W), jnp.float32),
        pltpu.VMEM((32, W), jnp.float32),
        pltpu.VMEM((W,), jnp.uint32),
        pltpu.VMEM((W,), jnp.uint32),
        pltpu.VMEM((2, L), jnp.float32),
        pltpu.SemaphoreType.DMA,
        pltpu.SemaphoreType.DMA,
    ],
)
def _sc_loss(x_hbm, t_hbm, mw_hbm, out_hbm,
             xb0, xb1, tb0, tb1, mb0, mb1, res, sem0, sem1):
    wid = lax.axis_index("s") * NC + lax.axis_index("c")
    base_row = wid * ROWS_PER_W
    base_band = wid * SC_CHUNKS
    xbs, tbs, mbs, sems = (xb0, xb1), (tb0, tb1), (mb0, mb1), (sem0, sem1)

    def start(g):
        slot = g % 2
        r0 = pl.multiple_of(base_row + g * 32, 32)
        ow = pl.multiple_of((base_band + g) * W, W)
        return [
            pltpu.async_copy(x_hbm.at[pl.ds(r0, 32), :], xbs[slot], sems[slot]),
            pltpu.async_copy(t_hbm.at[pl.ds(r0, 32), :], tbs[slot], sems[slot]),
            pltpu.async_copy(mw_hbm.at[pl.ds(ow, W)], mbs[slot], sems[slot]),
        ]

    acc = jnp.zeros((L,), jnp.float32)
    cnt = jnp.zeros((L,), jnp.float32)

    pend = start(0)
    for g in range(SC_CHUNKS):
        nxt = start(g + 1) if g + 1 < SC_CHUNKS else None
        for c in pend:
            c.wait()
        xb, tb, mb = xbs[g % 2], tbs[g % 2], mbs[g % 2]

        def body(i, carry, xb=xb, tb=tb, mb=mb):
            a, n = carry
            col = i * 16
            w = mb[pl.ds(col, 16)]
            for b in range(32):
                xg = xb[b, pl.ds(col, 16)]
                tg = tb[b, pl.ds(col, 16)]
                d = xg - tg
                ad = jnp.abs(d)
                c1 = jnp.minimum(ad, 1.0)
                loss = c1 * (ad - 0.5 * c1)
                mf = ((w >> jnp.uint32(b)) & jnp.uint32(1)).astype(jnp.float32)
                a = a + loss * mf
                n = n + mf
            return a, n

        acc, cnt = lax.fori_loop(0, GROUPS, body, (acc, cnt))
        pend = nxt

    res[0, :] = acc
    res[1, :] = cnt
    pltpu.sync_copy(res, out_hbm.at[wid])


TC_BLK = 512


def _tc_body(x_ref, t_ref, m_ref, sum_ref, cnt_ref):
    i = pl.program_id(0)

    @pl.when(i == 0)
    def _init():
        sum_ref[...] = jnp.zeros((1, W), jnp.float32)
        cnt_ref[...] = jnp.zeros((1, W), jnp.float32)

    d = x_ref[...] - t_ref[...]
    ad = jnp.abs(d)
    c1 = jnp.minimum(ad, 1.0)
    loss = c1 * (ad - 0.5 * c1)
    m = m_ref[...].astype(jnp.float32)
    sum_ref[...] += jnp.sum(loss * m, axis=0, keepdims=True)
    cnt_ref[...] += jnp.sum(m, axis=0, keepdims=True)


def _tc_loss(x, t, m):
    return pl.pallas_call(
        _tc_body,
        grid=(TC_ROWS // TC_BLK,),
        in_specs=[
            pl.BlockSpec((TC_BLK, W), lambda i: (i, 0)),
            pl.BlockSpec((TC_BLK, W), lambda i: (i, 0)),
            pl.BlockSpec((TC_BLK, W), lambda i: (i, 0)),
        ],
        out_specs=[
            pl.BlockSpec((1, W), lambda i: (0, 0)),
            pl.BlockSpec((1, W), lambda i: (0, 0)),
        ],
        out_shape=[
            jax.ShapeDtypeStruct((1, W), jnp.float32),
            jax.ShapeDtypeStruct((1, W), jnp.float32),
        ],
    )(x, t, m)


def kernel(inputs, targets, mask):
    x = inputs.reshape(ROWS, W)
    t = targets.reshape(ROWS, W)
    m = mask.reshape(ROWS, W)

    msc = m[TC_ROWS:].reshape(SC_ROWS // 32, 32, W).astype(jnp.uint32)
    mw = jnp.sum(msc << jnp.arange(32, dtype=jnp.uint32)[None, :, None], axis=1)

    sc_out = _sc_loss(x[TC_ROWS:], t[TC_ROWS:], mw.reshape(SC_ROWS // 32 * W))
    tc_sum, tc_cnt = _tc_loss(x[:TC_ROWS], t[:TC_ROWS], m[:TC_ROWS])

    s = jnp.sum(sc_out[:, 0, :]) + jnp.sum(tc_sum)
    c = jnp.sum(sc_out[:, 1, :]) + jnp.sum(tc_cnt)
    return s / jnp.maximum(c, 1.0)
